# speedup vs baseline: 2.2194x; 2.2194x over previous
import functools

import jax
import jax.numpy as jnp
from jax.experimental import pallas as pl

_EPS = 1e-5


def _ln_proj_body(x_ref, g_ref, b_ref, w_ref, bias_ref, o_ref, *, use_ln):
    x = x_ref[:]
    if use_ln:
        m = jnp.mean(x, axis=-1, keepdims=True)
        v = jnp.mean(x * x, axis=-1, keepdims=True) - m * m
        x = (x - m) * jax.lax.rsqrt(v + _EPS) * g_ref[:] + b_ref[:]
    o_ref[:] = jnp.dot(x, w_ref[:], preferred_element_type=jnp.float32) + bias_ref[:]


def _ln_proj(x, g, b, w, bias, *, use_ln, blk):
    rows, d = x.shape
    h = w.shape[1]
    grid = pl.cdiv(rows, blk)
    body = functools.partial(_ln_proj_body, use_ln=use_ln)
    return pl.pallas_call(
        body,
        grid=(grid,),
        in_specs=[
            pl.BlockSpec((blk, d), lambda i: (i, 0)),
            pl.BlockSpec((1, d), lambda i: (0, 0)),
            pl.BlockSpec((1, d), lambda i: (0, 0)),
            pl.BlockSpec((d, h), lambda i: (0, 0)),
            pl.BlockSpec((1, h), lambda i: (0, 0)),
        ],
        out_specs=pl.BlockSpec((blk, h), lambda i: (i, 0)),
        out_shape=jax.ShapeDtypeStruct((rows, h), jnp.float32),
    )(x, g.reshape(1, d), b.reshape(1, d), w, bias.reshape(1, h))


def kernel(node_embeddings, edge_embeddings, question_emb,
           node_norm_g, node_norm_b, rel_norm_g, rel_norm_b,
           node_W, node_b, rel_W, rel_b, q_W, q_b):
    node_tokens = _ln_proj(node_embeddings, node_norm_g, node_norm_b,
                           node_W, node_b, use_ln=True, blk=2048)
    relation_tokens = _ln_proj(edge_embeddings, rel_norm_g, rel_norm_b,
                               rel_W, rel_b, use_ln=True, blk=2048)
    question_tokens = _ln_proj(question_emb, rel_norm_g, rel_norm_b,
                               q_W, q_b, use_ln=False, blk=1024)
    return (node_tokens, relation_tokens, question_tokens)


# speedup vs baseline: 2.7816x; 1.2533x over previous
import functools

import jax
import jax.numpy as jnp
from jax.experimental import pallas as pl

_EPS = 1e-5


def _ln_proj_body(x_ref, g_ref, b_ref, w_ref, bias_ref, o_ref, *, use_ln):
    x = x_ref[:]
    if use_ln:
        m = jnp.mean(x, axis=-1, keepdims=True)
        v = jnp.mean(x * x, axis=-1, keepdims=True) - m * m
        x = (x - m) * jax.lax.rsqrt(v + _EPS) * g_ref[:] + b_ref[:]
    y = jnp.dot(x.astype(jnp.bfloat16), w_ref[:].astype(jnp.bfloat16),
                preferred_element_type=jnp.float32)
    o_ref[:] = y + bias_ref[:]


def _ln_proj(x, g, b, w, bias, *, use_ln, blk):
    rows, d = x.shape
    h = w.shape[1]
    grid = pl.cdiv(rows, blk)
    body = functools.partial(_ln_proj_body, use_ln=use_ln)
    return pl.pallas_call(
        body,
        grid=(grid,),
        in_specs=[
            pl.BlockSpec((blk, d), lambda i: (i, 0)),
            pl.BlockSpec((1, d), lambda i: (0, 0)),
            pl.BlockSpec((1, d), lambda i: (0, 0)),
            pl.BlockSpec((d, h), lambda i: (0, 0)),
            pl.BlockSpec((1, h), lambda i: (0, 0)),
        ],
        out_specs=pl.BlockSpec((blk, h), lambda i: (i, 0)),
        out_shape=jax.ShapeDtypeStruct((rows, h), jnp.float32),
    )(x, g.reshape(1, d), b.reshape(1, d), w, bias.reshape(1, h))


def kernel(node_embeddings, edge_embeddings, question_emb,
           node_norm_g, node_norm_b, rel_norm_g, rel_norm_b,
           node_W, node_b, rel_W, rel_b, q_W, q_b):
    node_tokens = _ln_proj(node_embeddings, node_norm_g, node_norm_b,
                           node_W, node_b, use_ln=True, blk=4096)
    relation_tokens = _ln_proj(edge_embeddings, rel_norm_g, rel_norm_b,
                               rel_W, rel_b, use_ln=True, blk=4096)
    question_tokens = _ln_proj(question_emb, rel_norm_g, rel_norm_b,
                               q_W, q_b, use_ln=False, blk=1024)
    return (node_tokens, relation_tokens, question_tokens)


# speedup vs baseline: 2.8970x; 1.0415x over previous
import functools

import jax
import jax.numpy as jnp
from jax.experimental import pallas as pl

_EPS = 1e-5


def _ln_proj_body(x_ref, w_ref, bias_ref, o_ref, *, use_ln):
    x = x_ref[:]
    if use_ln:
        m = jnp.mean(x, axis=-1, keepdims=True)
        c = x - m
        v = jnp.mean(c * c, axis=-1, keepdims=True)
        x = c * jax.lax.rsqrt(v + _EPS)
    y = jnp.dot(x.astype(jnp.bfloat16), w_ref[:],
                preferred_element_type=jnp.float32)
    o_ref[:] = y + bias_ref[:]


def _ln_proj(x, w_bf16, bias2, *, use_ln, blk):
    rows, d = x.shape
    h = w_bf16.shape[1]
    grid = pl.cdiv(rows, blk)
    body = functools.partial(_ln_proj_body, use_ln=use_ln)
    return pl.pallas_call(
        body,
        grid=(grid,),
        in_specs=[
            pl.BlockSpec((blk, d), lambda i: (i, 0)),
            pl.BlockSpec((d, h), lambda i: (0, 0)),
            pl.BlockSpec((1, h), lambda i: (0, 0)),
        ],
        out_specs=pl.BlockSpec((blk, h), lambda i: (i, 0)),
        out_shape=jax.ShapeDtypeStruct((rows, h), jnp.float32),
    )(x, w_bf16, bias2.reshape(1, h))


def kernel(node_embeddings, edge_embeddings, question_emb,
           node_norm_g, node_norm_b, rel_norm_g, rel_norm_b,
           node_W, node_b, rel_W, rel_b, q_W, q_b):
    node_Wg = (node_norm_g[:, None] * node_W).astype(jnp.bfloat16)
    node_bias2 = node_norm_b @ node_W + node_b
    rel_Wg = (rel_norm_g[:, None] * rel_W).astype(jnp.bfloat16)
    rel_bias2 = rel_norm_b @ rel_W + rel_b

    node_tokens = _ln_proj(node_embeddings, node_Wg, node_bias2,
                           use_ln=True, blk=4096)
    relation_tokens = _ln_proj(edge_embeddings, rel_Wg, rel_bias2,
                               use_ln=True, blk=4096)
    question_tokens = _ln_proj(question_emb, q_W.astype(jnp.bfloat16), q_b,
                               use_ln=False, blk=1024)
    return (node_tokens, relation_tokens, question_tokens)


# speedup vs baseline: 2.8981x; 1.0004x over previous
import functools

import jax
import jax.numpy as jnp
from jax.experimental import pallas as pl
from jax.experimental.pallas import tpu as pltpu

_EPS = 1e-5


def _ln_proj_body(x_ref, w_ref, bias_ref, o_ref, *, use_ln):
    x = x_ref[:]
    if use_ln:
        m = jnp.mean(x, axis=-1, keepdims=True)
        c = x - m
        v = jnp.mean(c * c, axis=-1, keepdims=True)
        x = c * jax.lax.rsqrt(v + _EPS)
    y = jnp.dot(x.astype(jnp.bfloat16), w_ref[:],
                preferred_element_type=jnp.float32)
    o_ref[:] = y + bias_ref[:]


def _ln_proj(x, w_bf16, bias2, *, use_ln, blk):
    rows, d = x.shape
    h = w_bf16.shape[1]
    grid = pl.cdiv(rows, blk)
    body = functools.partial(_ln_proj_body, use_ln=use_ln)
    return pl.pallas_call(
        body,
        grid=(grid,),
        in_specs=[
            pl.BlockSpec((blk, d), lambda i: (i, 0)),
            pl.BlockSpec((d, h), lambda i: (0, 0)),
            pl.BlockSpec((1, h), lambda i: (0, 0)),
        ],
        out_specs=pl.BlockSpec((blk, h), lambda i: (i, 0)),
        out_shape=jax.ShapeDtypeStruct((rows, h), jnp.float32),
        compiler_params=pltpu.CompilerParams(
            dimension_semantics=("parallel",)),
    )(x, w_bf16, bias2.reshape(1, h))


def kernel(node_embeddings, edge_embeddings, question_emb,
           node_norm_g, node_norm_b, rel_norm_g, rel_norm_b,
           node_W, node_b, rel_W, rel_b, q_W, q_b):
    node_Wg = (node_norm_g[:, None] * node_W).astype(jnp.bfloat16)
    node_bias2 = node_norm_b @ node_W + node_b
    rel_Wg = (rel_norm_g[:, None] * rel_W).astype(jnp.bfloat16)
    rel_bias2 = rel_norm_b @ rel_W + rel_b

    node_tokens = _ln_proj(node_embeddings, node_Wg, node_bias2,
                           use_ln=True, blk=4096)
    relation_tokens = _ln_proj(edge_embeddings, rel_Wg, rel_bias2,
                               use_ln=True, blk=4096)
    question_tokens = _ln_proj(question_emb, q_W.astype(jnp.bfloat16), q_b,
                               use_ln=False, blk=1024)
    return (node_tokens, relation_tokens, question_tokens)


# speedup vs baseline: 3.3690x; 1.1625x over previous
import functools

import jax
import jax.numpy as jnp
from jax.experimental import pallas as pl
from jax.experimental.pallas import tpu as pltpu

_EPS = 1e-5


def _ln_proj_body(x_ref, w_ref, bias_ref, o_ref, *, use_ln):
    x = x_ref[:]
    if use_ln:
        m = jnp.mean(x, axis=-1, keepdims=True)
        c = x - m
        v = jnp.mean(c * c, axis=-1, keepdims=True)
        x = c * jax.lax.rsqrt(v + _EPS)
    y = jnp.dot(x.astype(jnp.bfloat16), w_ref[:],
                preferred_element_type=jnp.float32)
    o_ref[:] = y + bias_ref[:]


def _ln_proj(x, w_bf16, bias2, *, use_ln, blk):
    rows, d = x.shape
    h = w_bf16.shape[1]
    grid = pl.cdiv(rows, blk)
    body = functools.partial(_ln_proj_body, use_ln=use_ln)
    return pl.pallas_call(
        body,
        grid=(grid,),
        in_specs=[
            pl.BlockSpec((blk, d), lambda i: (i, 0)),
            pl.BlockSpec((d, h), lambda i: (0, 0)),
            pl.BlockSpec((1, h), lambda i: (0, 0)),
        ],
        out_specs=pl.BlockSpec((blk, h), lambda i: (i, 0)),
        out_shape=jax.ShapeDtypeStruct((rows, h), jnp.float32),
        compiler_params=pltpu.CompilerParams(
            dimension_semantics=("parallel",)),
    )(x, w_bf16, bias2.reshape(1, h))


def kernel(node_embeddings, edge_embeddings, question_emb,
           node_norm_g, node_norm_b, rel_norm_g, rel_norm_b,
           node_W, node_b, rel_W, rel_b, q_W, q_b):
    node_Wg = (node_norm_g[:, None] * node_W).astype(jnp.bfloat16)
    node_bias2 = node_norm_b @ node_W + node_b
    rel_Wg = (rel_norm_g[:, None] * rel_W).astype(jnp.bfloat16)
    rel_bias2 = rel_norm_b @ rel_W + rel_b

    node_tokens = _ln_proj(node_embeddings, node_Wg, node_bias2,
                           use_ln=True, blk=8192)
    relation_tokens = _ln_proj(edge_embeddings, rel_Wg, rel_bias2,
                               use_ln=True, blk=8192)
    question_tokens = _ln_proj(question_emb, q_W.astype(jnp.bfloat16), q_b,
                               use_ln=False, blk=1024)
    return (node_tokens, relation_tokens, question_tokens)


# speedup vs baseline: 3.7426x; 1.1109x over previous
import functools

import jax
import jax.numpy as jnp
from jax.experimental import pallas as pl
from jax.experimental.pallas import tpu as pltpu

_EPS = 1e-5


def _ln_proj_body(x_ref, w_ref, bias_ref, o_ref, *, use_ln):
    x = x_ref[:]
    if use_ln:
        m = jnp.mean(x, axis=-1, keepdims=True)
        c = x - m
        v = jnp.mean(c * c, axis=-1, keepdims=True)
        x = c * jax.lax.rsqrt(v + _EPS)
    y = jnp.dot(x.astype(jnp.bfloat16), w_ref[:],
                preferred_element_type=jnp.float32)
    o_ref[:] = y + bias_ref[:]


def _ln_proj(x, w_bf16, bias2, *, use_ln, blk):
    rows, d = x.shape
    h = w_bf16.shape[1]
    grid = pl.cdiv(rows, blk)
    body = functools.partial(_ln_proj_body, use_ln=use_ln)
    return pl.pallas_call(
        body,
        grid=(grid,),
        in_specs=[
            pl.BlockSpec((blk, d), lambda i: (i, 0)),
            pl.BlockSpec((d, h), lambda i: (0, 0)),
            pl.BlockSpec((1, h), lambda i: (0, 0)),
        ],
        out_specs=pl.BlockSpec((blk, h), lambda i: (i, 0)),
        out_shape=jax.ShapeDtypeStruct((rows, h), jnp.float32),
        compiler_params=pltpu.CompilerParams(
            dimension_semantics=("parallel",)),
    )(x, w_bf16, bias2.reshape(1, h))


def kernel(node_embeddings, edge_embeddings, question_emb,
           node_norm_g, node_norm_b, rel_norm_g, rel_norm_b,
           node_W, node_b, rel_W, rel_b, q_W, q_b):
    node_Wg = (node_norm_g[:, None] * node_W).astype(jnp.bfloat16)
    node_bias2 = node_norm_b @ node_W + node_b
    rel_Wg = (rel_norm_g[:, None] * rel_W).astype(jnp.bfloat16)
    rel_bias2 = rel_norm_b @ rel_W + rel_b

    node_tokens = _ln_proj(node_embeddings, node_Wg, node_bias2,
                           use_ln=True, blk=5000)
    relation_tokens = _ln_proj(edge_embeddings, rel_Wg, rel_bias2,
                               use_ln=True, blk=16000)
    question_tokens = _ln_proj(question_emb, q_W.astype(jnp.bfloat16), q_b,
                               use_ln=False, blk=1024)
    return (node_tokens, relation_tokens, question_tokens)


# speedup vs baseline: 3.7991x; 1.0151x over previous
import functools

import jax
import jax.numpy as jnp
from jax.experimental import pallas as pl
from jax.experimental.pallas import tpu as pltpu

_EPS = 1e-5


def _ln_proj_body(x_ref, w_ref, bias_ref, o_ref, *, use_ln):
    x = x_ref[:]
    if use_ln:
        m = jnp.mean(x, axis=-1, keepdims=True)
        c = x - m
        v = jnp.mean(c * c, axis=-1, keepdims=True)
        x = c * jax.lax.rsqrt(v + _EPS)
    y = jnp.dot(x.astype(jnp.bfloat16), w_ref[:],
                preferred_element_type=jnp.float32)
    o_ref[:] = y + bias_ref[:]


def _ln_proj(x, w_bf16, bias2, *, use_ln, blk):
    rows, d = x.shape
    h = w_bf16.shape[1]
    grid = pl.cdiv(rows, blk)
    body = functools.partial(_ln_proj_body, use_ln=use_ln)
    return pl.pallas_call(
        body,
        grid=(grid,),
        in_specs=[
            pl.BlockSpec((blk, d), lambda i: (i, 0)),
            pl.BlockSpec((d, h), lambda i: (0, 0)),
            pl.BlockSpec((1, h), lambda i: (0, 0)),
        ],
        out_specs=pl.BlockSpec((blk, h), lambda i: (i, 0)),
        out_shape=jax.ShapeDtypeStruct((rows, h), jnp.float32),
        compiler_params=pltpu.CompilerParams(
            dimension_semantics=("parallel",)),
    )(x, w_bf16, bias2.reshape(1, h))


def kernel(node_embeddings, edge_embeddings, question_emb,
           node_norm_g, node_norm_b, rel_norm_g, rel_norm_b,
           node_W, node_b, rel_W, rel_b, q_W, q_b):
    node_Wg = (node_norm_g[:, None] * node_W).astype(jnp.bfloat16)
    node_bias2 = node_norm_b @ node_W + node_b
    rel_Wg = (rel_norm_g[:, None] * rel_W).astype(jnp.bfloat16)
    rel_bias2 = rel_norm_b @ rel_W + rel_b

    node_tokens = _ln_proj(node_embeddings, node_Wg, node_bias2,
                           use_ln=True, blk=5000)
    relation_tokens = _ln_proj(edge_embeddings, rel_Wg, rel_bias2,
                               use_ln=True, blk=20000)
    question_tokens = _ln_proj(question_emb, q_W.astype(jnp.bfloat16), q_b,
                               use_ln=False, blk=1024)
    return (node_tokens, relation_tokens, question_tokens)
